# baseline (speedup 1.0000x reference)
import functools

import jax
import jax.numpy as jnp
from jax import lax
from jax.experimental import pallas as pl
from jax.experimental.pallas import tpu as pltpu
from jax.experimental.pallas import tpu_sc as plsc

N = 10000
E = 320000
F_IN = 128
NB = 5
NBQ = 20
NC, NS, L = 2, 16, 16
NW = NC * NS
EC = E // NW
NSL = N // NS
G = 80
NG = EC // G
NP = 10240
BLKC = 256
BLKE = 512
f32 = jnp.float32
i32 = jnp.int32

_MESH = plsc.VectorSubcoreMesh(core_axis_name="c", subcore_axis_name="s",
                               num_cores=NC, num_subcores=NS)


def _phase_a(x, W1r, A1Sq, A1Dq):
    def body(x_ref, w_ref, as_ref, ad_ref, t1_ref, es_ref, ed_ref):
        h = jnp.dot(x_ref[...], w_ref[0], preferred_element_type=f32)
        t1_ref[0] = h
        es_ref[0] = jnp.dot(h, as_ref[0], preferred_element_type=f32)
        ed_ref[0] = jnp.dot(h, ad_ref[0], preferred_element_type=f32)

    return pl.pallas_call(
        body,
        grid=(NBQ,),
        in_specs=[
            pl.BlockSpec((N, F_IN), lambda k: (0, 0)),
            pl.BlockSpec((1, F_IN, 128), lambda k: (k, 0, 0)),
            pl.BlockSpec((1, 128, 2), lambda k: (k, 0, 0)),
            pl.BlockSpec((1, 128, 2), lambda k: (k, 0, 0)),
        ],
        out_specs=[
            pl.BlockSpec((1, N, 128), lambda k: (k, 0, 0)),
            pl.BlockSpec((1, N, 2), lambda k: (k, 0, 0)),
            pl.BlockSpec((1, N, 2), lambda k: (k, 0, 0)),
        ],
        out_shape=[
            jax.ShapeDtypeStruct((NBQ, N, 128), f32),
            jax.ShapeDtypeStruct((NBQ, N, 2), f32),
            jax.ShapeDtypeStruct((NBQ, N, 2), f32),
        ],
    )(x, W1r, A1Sq, A1Dq)


def _sc_pass1(ES, ED, ei, attrT, nph, hh_n, q_per_b):
    scratch = [
        pltpu.VMEM((EC,), i32),
        pltpu.VMEM((EC,), i32),
        pltpu.VMEM((EC,), i32),
        pltpu.VMEM((hh_n * N,), f32),
        pltpu.VMEM((hh_n * N,), f32),
    ]
    for _ in range(hh_n):
        scratch.append(pltpu.VMEM((EC,), f32))
    scratch.append(pltpu.VMEM((hh_n * N,), f32))

    @functools.partial(
        pl.kernel,
        out_type=[
            jax.ShapeDtypeStruct((nph, hh_n, E), f32),
            jax.ShapeDtypeStruct((nph, NW, hh_n * NP), f32),
        ],
        mesh=_MESH,
        scratch_types=scratch,
        compiler_params=pltpu.CompilerParams(use_tc_tiling_on_sc=False, needs_layout_passes=False),
    )
    def k(es_hbm, ed_hbm, ei_hbm, at_hbm, ex_hbm, den_hbm, *bufs):
        srcv, dstv, attv, esv, edv = bufs[:5]
        exv = bufs[5:5 + hh_n]
        denv = bufs[5 + hh_n]
        c = lax.axis_index("c")
        s = lax.axis_index("s")
        wid = c * NS + s
        eb = wid * EC
        pltpu.sync_copy(ei_hbm.at[0, pl.ds(eb, EC)], srcv)
        pltpu.sync_copy(ei_hbm.at[1, pl.ds(eb, EC)], dstv)
        zero = jnp.zeros((L,), f32)

        def phase(ph, _):
            b = ph // q_per_b
            pltpu.sync_copy(at_hbm.at[b, pl.ds(eb, EC)], attv)
            pltpu.sync_copy(es_hbm.at[ph], esv)
            pltpu.sync_copy(ed_hbm.at[ph], edv)

            def zbody(i, _):
                denv[pl.ds(i * L, L)] = zero
                return 0

            lax.fori_loop(0, hh_n * N // L, zbody, 0)

            def ebody(i, _):
                off = i * L
                sidx = srcv[pl.ds(off, L)]
                didx = dstv[pl.ds(off, L)]
                m = attv[pl.ds(off, L)] == 1
                for hh in range(hh_n):
                    esg = plsc.load_gather(esv, [sidx + hh * N])
                    edg = plsc.load_gather(edv, [didx + hh * N])
                    e = esg + edg
                    e = jnp.where(e >= 0.0, e, e * 0.2)
                    ex = jnp.where(m, jnp.exp(e), 0.0)
                    exv[hh][pl.ds(off, L)] = ex
                    if hh_n == 1:
                        plsc.addupdate_scatter(denv, [didx], ex)
                    else:
                        plsc.addupdate_scatter(denv, [didx * hh_n + hh], ex)
                return 0

            lax.fori_loop(0, EC // L, ebody, 0)
            for hh in range(hh_n):
                pltpu.sync_copy(exv[hh], ex_hbm.at[ph, hh, pl.ds(eb, EC)])
            pltpu.sync_copy(denv, den_hbm.at[ph, wid, pl.ds(0, hh_n * N)])
            return 0

        lax.fori_loop(0, nph, phase, 0)

    return k(ES, ED, ei, attrT)


def _sc_pass2(Tf, ei, EX, zz, nph_outer, hh_n, trows):
    DW = 64
    VH = DW // L

    @functools.partial(
        pl.kernel,
        out_type=jax.ShapeDtypeStruct((NC, nph_outer * hh_n, NP, DW), f32),
        mesh=_MESH,
        scratch_types=[
            pltpu.VMEM((EC,), i32),
            pltpu.VMEM((EC,), i32),
            pltpu.VMEM((EC,), f32),
            pltpu.VMEM((G, DW), f32),
            pltpu.VMEM((G, DW), f32),
            pltpu.VMEM((G, DW), f32),
            pltpu.VMEM((G, DW), f32),
            pltpu.VMEM((125, DW), f32),
            pltpu.VMEM_SHARED((N, DW), f32),
            pltpu.SemaphoreType.DMA,
            pltpu.SemaphoreType.DMA,
            pltpu.SemaphoreType.DMA,
            pltpu.SemaphoreType.DMA,
        ],
        compiler_params=pltpu.CompilerParams(use_tc_tiling_on_sc=False, needs_layout_passes=False),
    )
    def k(t_hbm, ei_hbm, ex_hbm, zz_hbm, out_hbm,
          srcv, dstv, w0v, rb0, rb1, rb2, rb3, ztile, acc,
          rs0, rs1, rs2, rs3):
        rbufs = (rb0, rb1, rb2, rb3)
        rsems = (rs0, rs1, rs2, rs3)
        NBUF = 4
        NMACRO = (NG + NBUF - 1) // NBUF
        c = lax.axis_index("c")
        s = lax.axis_index("s")
        wid = c * NS + s
        eb = wid * EC
        nb0 = s * NSL
        pltpu.sync_copy(ei_hbm.at[0, pl.ds(eb, EC)], srcv)
        pltpu.sync_copy(ei_hbm.at[1, pl.ds(eb, EC)], dstv)
        pltpu.sync_copy(zz_hbm, ztile)
        for j in range(NSL // 125):
            pltpu.sync_copy(ztile, acc.at[pl.ds(nb0 + j * 125, 125)])
        plsc.subcore_barrier()

        def issue_gather(g, gb, gs, row_off):
            for kk in range(G // L):
                s16 = srcv[pl.ds(g * G + kk * L, L)]
                if hh_n == 2:
                    i16 = s16 * 2 + row_off
                else:
                    i16 = s16 + row_off
                pltpu.async_copy(t_hbm.at[i16], gb.at[pl.ds(kk * L, L)], gs)

        def drain_gather(gb, gs):
            for kk in range(G // L):
                pltpu.make_async_copy(
                    t_hbm.at[pl.ds(0, L)], gb.at[pl.ds(kk * L, L)], gs).wait()

        def drain_scatter(sb, ss):
            for kk in range(G // L):
                pltpu.make_async_copy(
                    sb.at[pl.ds(kk * L, L)], acc.at[pl.ds(0, L)], ss).wait()

        def process(g, gb, ss):
            base = g * G
            drain_gather(gb, ss)
            for kk in range(G // L):
                w0_16 = w0v[pl.ds(base + kk * L, L)]
                for rr in range(L):
                    r = kk * L + rr
                    w0 = w0_16[rr]
                    for v in range(VH):
                        gb[r, pl.ds(v * L, L)] = gb[r, pl.ds(v * L, L)] * w0
            for kk in range(G // L):
                d16 = dstv[pl.ds(base + kk * L, L)]
                pltpu.async_copy(gb.at[pl.ds(kk * L, L)], acc.at[d16], ss,
                                 add=True)

        def phase(ph, _):
            po = ph // hh_n
            hh = ph % hh_n
            row_off = po * (hh_n * trows) + hh
            pltpu.sync_copy(ex_hbm.at[po, hh, pl.ds(eb, EC)], w0v)
            for kb in range(NBUF):
                issue_gather(kb, rbufs[kb], rsems[kb], row_off)

            def macro(m, _):
                gbase = m * NBUF
                for kb in range(NBUF):
                    g = gbase + kb

                    @pl.when(g >= NBUF)
                    def _():
                        drain_scatter(rbufs[kb], rsems[kb])

                    @pl.when(jnp.logical_and(g >= NBUF, g < NG))
                    def _():
                        issue_gather(g, rbufs[kb], rsems[kb], row_off)
                for kb in range(NBUF):
                    g = gbase + kb

                    @pl.when(g < NG)
                    def _():
                        process(g, rbufs[kb], rsems[kb])

                return 0

            lax.fori_loop(0, NMACRO, macro, 0)
            for kb in range(NBUF):
                if (NMACRO - 1) * NBUF + kb < NG:
                    drain_scatter(rbufs[kb], rsems[kb])
            plsc.subcore_barrier()
            pltpu.sync_copy(acc.at[pl.ds(nb0, NSL)],
                            out_hbm.at[c, ph, pl.ds(nb0, NSL)])
            for j in range(NSL // 125):
                pltpu.sync_copy(ztile, acc.at[pl.ds(nb0 + j * 125, 125)])
            plsc.subcore_barrier()
            return 0

        lax.fori_loop(0, nph_outer * hh_n, phase, 0)

    return k(Tf, ei, EX, zz)


def _reduce_partials(DP, nph, inner):
    CH = 2048

    def body(dp_ref, out_ref):
        acc = dp_ref[:, 0]
        for kk in range(1, NW):
            acc = acc + dp_ref[:, kk]
        out_ref[...] = acc

    return pl.pallas_call(
        body,
        grid=(inner // CH,),
        in_specs=[pl.BlockSpec((nph, NW, CH), lambda i: (0, 0, i))],
        out_specs=pl.BlockSpec((nph, CH), lambda i: (0, i)),
        out_shape=jax.ShapeDtypeStruct((nph, inner), f32),
    )(DP)


def _phase_c(OUT1P, DEN1P, W2r, A2Sbd, A2Dbd):
    def body(p_ref, den_ref, w2_ref, a2s_ref, a2d_ref,
             t2_ref, es2_ref, ed2_ref):
        h2 = None
        for bq in range(NBQ):
            p0 = p_ref[0, 2 * bq] + p_ref[1, 2 * bq]
            p1 = p_ref[0, 2 * bq + 1] + p_ref[1, 2 * bq + 1]
            den = den_ref[bq]
            d0 = jnp.broadcast_to(den[:, 0:1], (BLKC, 64))
            d1 = jnp.broadcast_to(den[:, 1:2], (BLKC, 64))
            gacc = jnp.concatenate([p0 / (d0 + 1e-16), p1 / (d1 + 1e-16)],
                                   axis=1)
            gacc = jnp.where(gacc > 0, gacc, (jnp.exp(gacc) - 1.0))
            t = jnp.dot(gacc, w2_ref[bq], preferred_element_type=f32)
            h2 = t if h2 is None else h2 + t
        es2_ref[...] = jnp.dot(h2, a2s_ref[...], preferred_element_type=f32)
        ed2_ref[...] = jnp.dot(h2, a2d_ref[...], preferred_element_type=f32)
        for b in range(NB):
            t2_ref[b] = h2[:, b * 64:(b + 1) * 64]

    return pl.pallas_call(
        body,
        grid=(NP // BLKC,),
        in_specs=[
            pl.BlockSpec((NC, 2 * NBQ, BLKC, 64), lambda i: (0, 0, i, 0)),
            pl.BlockSpec((NBQ, BLKC, 2), lambda i: (0, i, 0)),
            pl.BlockSpec((NBQ, 128, NB * 64), lambda i: (0, 0, 0)),
            pl.BlockSpec((NB * 64, NB), lambda i: (0, 0)),
            pl.BlockSpec((NB * 64, NB), lambda i: (0, 0)),
        ],
        out_specs=[
            pl.BlockSpec((NB, BLKC, 64), lambda i: (0, i, 0)),
            pl.BlockSpec((BLKC, NB), lambda i: (i, 0)),
            pl.BlockSpec((BLKC, NB), lambda i: (i, 0)),
        ],
        out_shape=[
            jax.ShapeDtypeStruct((NB, NP, 64), f32),
            jax.ShapeDtypeStruct((NP, NB), f32),
            jax.ShapeDtypeStruct((NP, NB), f32),
        ],
    )(OUT1P, DEN1P, W2r, A2Sbd, A2Dbd)


def _phase_e(OUT2P, DEN2P, Wf, bf):
    def body(p_ref, den_ref, wf_ref, bf_ref, out_ref):
        ys = []
        for b in range(NB):
            p = p_ref[0, b] + p_ref[1, b]
            den = den_ref[b]
            d = jnp.broadcast_to(den, (BLKE, 64))
            y = p / (d + 1e-16)
            ys.append(jnp.where(y > 0, y, (jnp.exp(y) - 1.0)))
        y = jnp.concatenate(ys, axis=1)
        o = jnp.dot(y, wf_ref[...], preferred_element_type=f32) + bf_ref[...]
        out_ref[...] = jnp.where(o > 0, o, (jnp.exp(o) - 1.0))

    return pl.pallas_call(
        body,
        grid=(NP // BLKE,),
        in_specs=[
            pl.BlockSpec((NC, NB, BLKE, 64), lambda i: (0, 0, i, 0)),
            pl.BlockSpec((NB, BLKE, 1), lambda i: (0, i, 0)),
            pl.BlockSpec((NB * 64, 64), lambda i: (0, 0)),
            pl.BlockSpec((1, 64), lambda i: (0, 0)),
        ],
        out_specs=pl.BlockSpec((BLKE, 64), lambda i: (i, 0)),
        out_shape=jax.ShapeDtypeStruct((NP, 64), f32),
    )(OUT2P, DEN2P, Wf, bf)


def kernel(x, edge_index, edge_attr, W1, a1s, a1d, W2, a2s, a2d, Wf, bf):
    ei = edge_index.astype(i32)
    attrT = jnp.concatenate([edge_attr.astype(i32).T,
                             jnp.ones((1, E), i32)], axis=0)

    W1r = W1.reshape(NB, F_IN, 4, 128).transpose(0, 2, 1, 3)
    W1r = W1r.reshape(NBQ, F_IN, 128)
    a1s_r = a1s.reshape(NBQ, 2, 64)
    a1d_r = a1d.reshape(NBQ, 2, 64)
    A1Sq = jnp.zeros((NBQ, 128, 2), f32)
    A1Sq = A1Sq.at[:, 0:64, 0].set(a1s_r[:, 0, :])
    A1Sq = A1Sq.at[:, 64:128, 1].set(a1s_r[:, 1, :])
    A1Dq = jnp.zeros((NBQ, 128, 2), f32)
    A1Dq = A1Dq.at[:, 0:64, 0].set(a1d_r[:, 0, :])
    A1Dq = A1Dq.at[:, 64:128, 1].set(a1d_r[:, 1, :])
    W2r = jnp.zeros((NBQ, 128, NB * 64), f32)
    for b in range(NB):
        for q in range(4):
            W2r = W2r.at[b * 4 + q, :, b * 64:(b + 1) * 64].set(
                W2[b, q * 128:(q + 1) * 128, :])
    A2Sbd = jnp.zeros((NB * 64, NB), f32)
    A2Dbd = jnp.zeros((NB * 64, NB), f32)
    for b in range(NB):
        A2Sbd = A2Sbd.at[b * 64:(b + 1) * 64, b].set(a2s[b, 0, :])
        A2Dbd = A2Dbd.at[b * 64:(b + 1) * 64, b].set(a2d[b, 0, :])
    zz64 = jnp.zeros((125, 64), f32)

    T1, ES1, ED1 = _phase_a(x, W1r, A1Sq, A1Dq)
    ES1T = jnp.transpose(ES1, (0, 2, 1)).reshape(NBQ, 2 * N)
    ED1T = jnp.transpose(ED1, (0, 2, 1)).reshape(NBQ, 2 * N)
    EX1, DEN1P = _sc_pass1(ES1T, ED1T, ei, attrT, NBQ, 2, 4)
    OUT1P = _sc_pass2(T1.reshape(NBQ * N * 2, 64), ei, EX1, zz64, NBQ, 2, N)
    DEN1S = _reduce_partials(DEN1P, NBQ, 2 * NP).reshape(NBQ, NP, 2)

    T2, ES2, ED2 = _phase_c(OUT1P, DEN1S, W2r, A2Sbd, A2Dbd)
    EX2, DEN2P = _sc_pass1(ES2.T[:, :N], ED2.T[:, :N], ei, attrT, NB, 1, 1)
    OUT2P = _sc_pass2(T2.reshape(NB * NP, 64), ei, EX2, zz64, NB, 1, NP)
    DEN2S = _reduce_partials(DEN2P, NB, NP).reshape(NB, NP, 1)

    return _phase_e(OUT2P, DEN2S, Wf, bf.reshape(1, 64))[:N]


# speedup vs baseline: 1.2775x; 1.2775x over previous
import functools

import jax
import jax.numpy as jnp
from jax import lax
from jax.experimental import pallas as pl
from jax.experimental.pallas import tpu as pltpu
from jax.experimental.pallas import tpu_sc as plsc

N = 10000
E = 320000
F_IN = 128
NB = 5
NBQ = 20
NC, NS, L = 2, 16, 16
NW = NC * NS
EC = E // NW
NSL = N // NS
G = 80
NG = EC // G
NP = 10240
BLKC = 256
BLKE = 512
f32 = jnp.float32
i32 = jnp.int32

_MESH = plsc.VectorSubcoreMesh(core_axis_name="c", subcore_axis_name="s",
                               num_cores=NC, num_subcores=NS)


def _phase_a(x, W1r, A1Sq, A1Dq):
    def body(x_ref, w_ref, as_ref, ad_ref, t1_ref, es_ref, ed_ref):
        h = jnp.dot(x_ref[...], w_ref[0], preferred_element_type=f32)
        t1_ref[0] = h
        es_ref[0] = jnp.dot(h, as_ref[0], preferred_element_type=f32)
        ed_ref[0] = jnp.dot(h, ad_ref[0], preferred_element_type=f32)

    return pl.pallas_call(
        body,
        grid=(NBQ,),
        in_specs=[
            pl.BlockSpec((N, F_IN), lambda k: (0, 0)),
            pl.BlockSpec((1, F_IN, 128), lambda k: (k, 0, 0)),
            pl.BlockSpec((1, 128, 2), lambda k: (k, 0, 0)),
            pl.BlockSpec((1, 128, 2), lambda k: (k, 0, 0)),
        ],
        out_specs=[
            pl.BlockSpec((1, N, 128), lambda k: (k, 0, 0)),
            pl.BlockSpec((1, N, 2), lambda k: (k, 0, 0)),
            pl.BlockSpec((1, N, 2), lambda k: (k, 0, 0)),
        ],
        out_shape=[
            jax.ShapeDtypeStruct((NBQ, N, 128), f32),
            jax.ShapeDtypeStruct((NBQ, N, 2), f32),
            jax.ShapeDtypeStruct((NBQ, N, 2), f32),
        ],
    )(x, W1r, A1Sq, A1Dq)


def _sc_pass1(ES, ED, ei, attrT, nph, hh_n, q_per_b):
    scratch = [
        pltpu.VMEM((EC,), i32),
        pltpu.VMEM((EC,), i32),
        pltpu.VMEM((EC,), i32),
        pltpu.VMEM((hh_n * N,), f32),
        pltpu.VMEM((hh_n * N,), f32),
    ]
    for _ in range(hh_n):
        scratch.append(pltpu.VMEM((EC,), f32))
    scratch.append(pltpu.VMEM((hh_n * N,), f32))

    @functools.partial(
        pl.kernel,
        out_type=[
            jax.ShapeDtypeStruct((nph, hh_n, E), f32),
            jax.ShapeDtypeStruct((nph, NW, hh_n * NP), f32),
        ],
        mesh=_MESH,
        scratch_types=scratch,
        compiler_params=pltpu.CompilerParams(use_tc_tiling_on_sc=False, needs_layout_passes=False),
    )
    def k(es_hbm, ed_hbm, ei_hbm, at_hbm, ex_hbm, den_hbm, *bufs):
        srcv, dstv, attv, esv, edv = bufs[:5]
        exv = bufs[5:5 + hh_n]
        denv = bufs[5 + hh_n]
        c = lax.axis_index("c")
        s = lax.axis_index("s")
        wid = c * NS + s
        eb = wid * EC
        pltpu.sync_copy(ei_hbm.at[0, pl.ds(eb, EC)], srcv)
        pltpu.sync_copy(ei_hbm.at[1, pl.ds(eb, EC)], dstv)
        zero = jnp.zeros((L,), f32)

        def phase(ph, _):
            b = ph // q_per_b
            pltpu.sync_copy(at_hbm.at[b, pl.ds(eb, EC)], attv)
            pltpu.sync_copy(es_hbm.at[ph], esv)
            pltpu.sync_copy(ed_hbm.at[ph], edv)

            def zbody(i, _):
                denv[pl.ds(i * L, L)] = zero
                return 0

            lax.fori_loop(0, hh_n * N // L, zbody, 0)

            def ebody(i, _):
                off = i * L
                sidx = srcv[pl.ds(off, L)]
                didx = dstv[pl.ds(off, L)]
                m = attv[pl.ds(off, L)] == 1
                for hh in range(hh_n):
                    esg = plsc.load_gather(esv, [sidx + hh * N])
                    edg = plsc.load_gather(edv, [didx + hh * N])
                    e = esg + edg
                    e = jnp.where(e >= 0.0, e, e * 0.2)
                    ex = jnp.where(m, jnp.exp(e), 0.0)
                    exv[hh][pl.ds(off, L)] = ex
                    if hh_n == 1:
                        plsc.addupdate_scatter(denv, [didx], ex)
                    else:
                        plsc.addupdate_scatter(denv, [didx * hh_n + hh], ex)
                return 0

            lax.fori_loop(0, EC // L, ebody, 0)
            for hh in range(hh_n):
                pltpu.sync_copy(exv[hh], ex_hbm.at[ph, hh, pl.ds(eb, EC)])
            pltpu.sync_copy(denv, den_hbm.at[ph, wid, pl.ds(0, hh_n * N)])
            return 0

        lax.fori_loop(0, nph, phase, 0)

    return k(ES, ED, ei, attrT)


def _sc_pass2(Tf, ei, EX, zz, nph_outer, hh_n, trows):
    DW = 64
    VH = DW // L

    @functools.partial(
        pl.kernel,
        out_type=jax.ShapeDtypeStruct((NC, nph_outer * hh_n, NP, DW), f32),
        mesh=_MESH,
        scratch_types=[
            pltpu.VMEM((EC,), i32),
            pltpu.VMEM((EC,), i32),
            pltpu.VMEM((EC,), f32),
            pltpu.VMEM((G, DW), f32),
            pltpu.VMEM((G, DW), f32),
            pltpu.VMEM((G, DW), f32),
            pltpu.VMEM((G, DW), f32),
            pltpu.VMEM((125, DW), f32),
            pltpu.VMEM_SHARED((N, DW), f32),
            pltpu.SemaphoreType.DMA,
            pltpu.SemaphoreType.DMA,
            pltpu.SemaphoreType.DMA,
            pltpu.SemaphoreType.DMA,
        ],
        compiler_params=pltpu.CompilerParams(use_tc_tiling_on_sc=False, needs_layout_passes=False),
    )
    def k(t_hbm, ei_hbm, ex_hbm, zz_hbm, out_hbm,
          srcv, dstv, w0v, rb0, rb1, rb2, rb3, ztile, acc,
          rs0, rs1, rs2, rs3):
        rbufs = (rb0, rb1, rb2, rb3)
        rsems = (rs0, rs1, rs2, rs3)
        NBUF = 4
        NMACRO = (NG + NBUF - 1) // NBUF
        c = lax.axis_index("c")
        s = lax.axis_index("s")
        wid = c * NS + s
        eb = wid * EC
        nb0 = s * NSL
        pltpu.sync_copy(ei_hbm.at[0, pl.ds(eb, EC)], srcv)
        pltpu.sync_copy(ei_hbm.at[1, pl.ds(eb, EC)], dstv)
        pltpu.sync_copy(zz_hbm, ztile)
        for j in range(NSL // 125):
            pltpu.sync_copy(ztile, acc.at[pl.ds(nb0 + j * 125, 125)])
        plsc.subcore_barrier()

        def issue_gather(g, gb, gs, row_off):
            for kk in range(G // L):
                s16 = srcv[pl.ds(g * G + kk * L, L)]
                if hh_n == 2:
                    i16 = s16 * 2 + row_off
                else:
                    i16 = s16 + row_off
                pltpu.async_copy(t_hbm.at[i16], gb.at[pl.ds(kk * L, L)], gs)

        def drain_gather(gb, gs):
            for kk in range(G // L):
                pltpu.make_async_copy(
                    t_hbm.at[pl.ds(0, L)], gb.at[pl.ds(kk * L, L)], gs).wait()

        def drain_scatter(sb, ss):
            for kk in range(G // L):
                pltpu.make_async_copy(
                    sb.at[pl.ds(kk * L, L)], acc.at[pl.ds(0, L)], ss).wait()

        def process(g, gb, ss):
            base = g * G
            drain_gather(gb, ss)
            for kk in range(G // L):
                w0_16 = w0v[pl.ds(base + kk * L, L)]
                for rr in range(L):
                    r = kk * L + rr
                    w0 = w0_16[rr]
                    for v in range(VH):
                        gb[r, pl.ds(v * L, L)] = gb[r, pl.ds(v * L, L)] * w0
            for kk in range(G // L):
                d16 = dstv[pl.ds(base + kk * L, L)]
                pltpu.async_copy(gb.at[pl.ds(kk * L, L)], acc.at[d16], ss,
                                 add=True)

        def phase(ph, _):
            po = ph // hh_n
            hh = ph % hh_n
            row_off = po * (hh_n * trows) + hh
            pltpu.sync_copy(ex_hbm.at[po, hh, pl.ds(eb, EC)], w0v)

            def slot_refill(g, kb):
                @pl.when(jnp.logical_and(g >= NBUF, g - NBUF < NG))
                def _():
                    drain_scatter(rbufs[kb], rsems[kb])

                @pl.when(g < NG)
                def _():
                    issue_gather(g, rbufs[kb], rsems[kb], row_off)

            def slot_process(g, kb):
                @pl.when(g < NG)
                def _():
                    process(g, rbufs[kb], rsems[kb])

            issue_gather(0, rbufs[0], rsems[0], row_off)
            issue_gather(1, rbufs[1], rsems[1], row_off)

            def macro(m, _):
                g4 = m * NBUF
                slot_refill(g4 + 2, 2)
                slot_refill(g4 + 3, 3)
                slot_process(g4, 0)
                slot_process(g4 + 1, 1)
                slot_refill(g4 + 4, 0)
                slot_refill(g4 + 5, 1)
                slot_process(g4 + 2, 2)
                slot_process(g4 + 3, 3)
                return 0

            lax.fori_loop(0, NMACRO, macro, 0)
            plsc.subcore_barrier()
            pltpu.sync_copy(acc.at[pl.ds(nb0, NSL)],
                            out_hbm.at[c, ph, pl.ds(nb0, NSL)])
            for j in range(NSL // 125):
                pltpu.sync_copy(ztile, acc.at[pl.ds(nb0 + j * 125, 125)])
            plsc.subcore_barrier()
            return 0

        lax.fori_loop(0, nph_outer * hh_n, phase, 0)

    return k(Tf, ei, EX, zz)


def _reduce_partials(DP, nph, inner):
    CH = 2048

    def body(dp_ref, out_ref):
        acc = dp_ref[:, 0]
        for kk in range(1, NW):
            acc = acc + dp_ref[:, kk]
        out_ref[...] = acc

    return pl.pallas_call(
        body,
        grid=(inner // CH,),
        in_specs=[pl.BlockSpec((nph, NW, CH), lambda i: (0, 0, i))],
        out_specs=pl.BlockSpec((nph, CH), lambda i: (0, i)),
        out_shape=jax.ShapeDtypeStruct((nph, inner), f32),
    )(DP)


def _phase_c(OUT1P, DEN1P, W2r, A2Sbd, A2Dbd):
    def body(p_ref, den_ref, w2_ref, a2s_ref, a2d_ref,
             t2_ref, es2_ref, ed2_ref):
        h2 = None
        for bq in range(NBQ):
            p0 = p_ref[0, 2 * bq] + p_ref[1, 2 * bq]
            p1 = p_ref[0, 2 * bq + 1] + p_ref[1, 2 * bq + 1]
            den = den_ref[bq]
            d0 = jnp.broadcast_to(den[:, 0:1], (BLKC, 64))
            d1 = jnp.broadcast_to(den[:, 1:2], (BLKC, 64))
            gacc = jnp.concatenate([p0 / (d0 + 1e-16), p1 / (d1 + 1e-16)],
                                   axis=1)
            gacc = jnp.where(gacc > 0, gacc, (jnp.exp(gacc) - 1.0))
            t = jnp.dot(gacc, w2_ref[bq], preferred_element_type=f32)
            h2 = t if h2 is None else h2 + t
        es2_ref[...] = jnp.dot(h2, a2s_ref[...], preferred_element_type=f32)
        ed2_ref[...] = jnp.dot(h2, a2d_ref[...], preferred_element_type=f32)
        for b in range(NB):
            t2_ref[b] = h2[:, b * 64:(b + 1) * 64]

    return pl.pallas_call(
        body,
        grid=(NP // BLKC,),
        in_specs=[
            pl.BlockSpec((NC, 2 * NBQ, BLKC, 64), lambda i: (0, 0, i, 0)),
            pl.BlockSpec((NBQ, BLKC, 2), lambda i: (0, i, 0)),
            pl.BlockSpec((NBQ, 128, NB * 64), lambda i: (0, 0, 0)),
            pl.BlockSpec((NB * 64, NB), lambda i: (0, 0)),
            pl.BlockSpec((NB * 64, NB), lambda i: (0, 0)),
        ],
        out_specs=[
            pl.BlockSpec((NB, BLKC, 64), lambda i: (0, i, 0)),
            pl.BlockSpec((BLKC, NB), lambda i: (i, 0)),
            pl.BlockSpec((BLKC, NB), lambda i: (i, 0)),
        ],
        out_shape=[
            jax.ShapeDtypeStruct((NB, NP, 64), f32),
            jax.ShapeDtypeStruct((NP, NB), f32),
            jax.ShapeDtypeStruct((NP, NB), f32),
        ],
    )(OUT1P, DEN1P, W2r, A2Sbd, A2Dbd)


def _phase_e(OUT2P, DEN2P, Wf, bf):
    def body(p_ref, den_ref, wf_ref, bf_ref, out_ref):
        ys = []
        for b in range(NB):
            p = p_ref[0, b] + p_ref[1, b]
            den = den_ref[b]
            d = jnp.broadcast_to(den, (BLKE, 64))
            y = p / (d + 1e-16)
            ys.append(jnp.where(y > 0, y, (jnp.exp(y) - 1.0)))
        y = jnp.concatenate(ys, axis=1)
        o = jnp.dot(y, wf_ref[...], preferred_element_type=f32) + bf_ref[...]
        out_ref[...] = jnp.where(o > 0, o, (jnp.exp(o) - 1.0))

    return pl.pallas_call(
        body,
        grid=(NP // BLKE,),
        in_specs=[
            pl.BlockSpec((NC, NB, BLKE, 64), lambda i: (0, 0, i, 0)),
            pl.BlockSpec((NB, BLKE, 1), lambda i: (0, i, 0)),
            pl.BlockSpec((NB * 64, 64), lambda i: (0, 0)),
            pl.BlockSpec((1, 64), lambda i: (0, 0)),
        ],
        out_specs=pl.BlockSpec((BLKE, 64), lambda i: (i, 0)),
        out_shape=jax.ShapeDtypeStruct((NP, 64), f32),
    )(OUT2P, DEN2P, Wf, bf)


def kernel(x, edge_index, edge_attr, W1, a1s, a1d, W2, a2s, a2d, Wf, bf):
    ei = edge_index.astype(i32)
    attrT = jnp.concatenate([edge_attr.astype(i32).T,
                             jnp.ones((1, E), i32)], axis=0)

    W1r = W1.reshape(NB, F_IN, 4, 128).transpose(0, 2, 1, 3)
    W1r = W1r.reshape(NBQ, F_IN, 128)
    a1s_r = a1s.reshape(NBQ, 2, 64)
    a1d_r = a1d.reshape(NBQ, 2, 64)
    A1Sq = jnp.zeros((NBQ, 128, 2), f32)
    A1Sq = A1Sq.at[:, 0:64, 0].set(a1s_r[:, 0, :])
    A1Sq = A1Sq.at[:, 64:128, 1].set(a1s_r[:, 1, :])
    A1Dq = jnp.zeros((NBQ, 128, 2), f32)
    A1Dq = A1Dq.at[:, 0:64, 0].set(a1d_r[:, 0, :])
    A1Dq = A1Dq.at[:, 64:128, 1].set(a1d_r[:, 1, :])
    W2r = jnp.zeros((NBQ, 128, NB * 64), f32)
    for b in range(NB):
        for q in range(4):
            W2r = W2r.at[b * 4 + q, :, b * 64:(b + 1) * 64].set(
                W2[b, q * 128:(q + 1) * 128, :])
    A2Sbd = jnp.zeros((NB * 64, NB), f32)
    A2Dbd = jnp.zeros((NB * 64, NB), f32)
    for b in range(NB):
        A2Sbd = A2Sbd.at[b * 64:(b + 1) * 64, b].set(a2s[b, 0, :])
        A2Dbd = A2Dbd.at[b * 64:(b + 1) * 64, b].set(a2d[b, 0, :])
    zz64 = jnp.zeros((125, 64), f32)

    T1, ES1, ED1 = _phase_a(x, W1r, A1Sq, A1Dq)
    ES1T = jnp.transpose(ES1, (0, 2, 1)).reshape(NBQ, 2 * N)
    ED1T = jnp.transpose(ED1, (0, 2, 1)).reshape(NBQ, 2 * N)
    EX1, DEN1P = _sc_pass1(ES1T, ED1T, ei, attrT, NBQ, 2, 4)
    OUT1P = _sc_pass2(T1.reshape(NBQ * N * 2, 64), ei, EX1, zz64, NBQ, 2, N)
    DEN1S = _reduce_partials(DEN1P, NBQ, 2 * NP).reshape(NBQ, NP, 2)

    T2, ES2, ED2 = _phase_c(OUT1P, DEN1S, W2r, A2Sbd, A2Dbd)
    EX2, DEN2P = _sc_pass1(ES2.T[:, :N], ED2.T[:, :N], ei, attrT, NB, 1, 1)
    OUT2P = _sc_pass2(T2.reshape(NB * NP, 64), ei, EX2, zz64, NB, 1, NP)
    DEN2S = _reduce_partials(DEN2P, NB, NP).reshape(NB, NP, 1)

    return _phase_e(OUT2P, DEN2S, Wf, bf.reshape(1, 64))[:N]


# speedup vs baseline: 1.4210x; 1.1123x over previous
import functools

import jax
import jax.numpy as jnp
from jax import lax
from jax.experimental import pallas as pl
from jax.experimental.pallas import tpu as pltpu
from jax.experimental.pallas import tpu_sc as plsc

N = 10000
E = 320000
F_IN = 128
NB = 5
NBQ = 20
NC, NS, L = 2, 16, 16
NW = NC * NS
EC = E // NW
ECP = EC + 128
NSL = N // NS
G = 80
NG = EC // G
NP = 10240
BLKC = 256
BLKE = 512
f32 = jnp.float32
i32 = jnp.int32

_MESH = plsc.VectorSubcoreMesh(core_axis_name="c", subcore_axis_name="s",
                               num_cores=NC, num_subcores=NS)


def _phase_a(x, W1r, A1Sq, A1Dq):
    def body(x_ref, w_ref, as_ref, ad_ref, t1_ref, es_ref, ed_ref):
        h = jnp.dot(x_ref[...], w_ref[0], preferred_element_type=f32)
        t1_ref[0] = h
        es_ref[0] = jnp.dot(h, as_ref[0], preferred_element_type=f32)
        ed_ref[0] = jnp.dot(h, ad_ref[0], preferred_element_type=f32)

    return pl.pallas_call(
        body,
        grid=(NBQ,),
        in_specs=[
            pl.BlockSpec((N, F_IN), lambda k: (0, 0)),
            pl.BlockSpec((1, F_IN, 128), lambda k: (k, 0, 0)),
            pl.BlockSpec((1, 128, 2), lambda k: (k, 0, 0)),
            pl.BlockSpec((1, 128, 2), lambda k: (k, 0, 0)),
        ],
        out_specs=[
            pl.BlockSpec((1, N, 128), lambda k: (k, 0, 0)),
            pl.BlockSpec((1, N, 2), lambda k: (k, 0, 0)),
            pl.BlockSpec((1, N, 2), lambda k: (k, 0, 0)),
        ],
        out_shape=[
            jax.ShapeDtypeStruct((NBQ, N, 128), f32),
            jax.ShapeDtypeStruct((NBQ, N, 2), f32),
            jax.ShapeDtypeStruct((NBQ, N, 2), f32),
        ],
    )(x, W1r, A1Sq, A1Dq)


def _sc_pass1(ES, ED, ei, attrT, nph, hh_n, q_per_b):
    scratch = [
        pltpu.VMEM((EC,), i32),
        pltpu.VMEM((EC,), i32),
        pltpu.VMEM((EC,), i32),
        pltpu.VMEM((hh_n * N,), f32),
        pltpu.VMEM((hh_n * N,), f32),
    ]
    for _ in range(hh_n):
        scratch.append(pltpu.VMEM((ECP,), f32))
    scratch.append(pltpu.VMEM((hh_n * N,), f32))
    scratch.append(pltpu.VMEM((ECP,), i32))
    scratch.append(pltpu.VMEM((ECP,), i32))
    scratch.append(pltpu.VMEM((L,), i32))

    @functools.partial(
        pl.kernel,
        out_type=[
            jax.ShapeDtypeStruct((nph, hh_n, NW * ECP), f32),
            jax.ShapeDtypeStruct((nph, NW, hh_n * NP), f32),
            jax.ShapeDtypeStruct((NB, NW * ECP), i32),
            jax.ShapeDtypeStruct((NB, NW * ECP), i32),
            jax.ShapeDtypeStruct((NB, NW, L), i32),
        ],
        mesh=_MESH,
        scratch_types=scratch,
        compiler_params=pltpu.CompilerParams(use_tc_tiling_on_sc=False, needs_layout_passes=False),
    )
    def k(es_hbm, ed_hbm, ei_hbm, at_hbm,
          ex_hbm, den_hbm, srcc_hbm, dstc_hbm, cnt_hbm, *bufs):
        srcv, dstv, attv, esv, edv = bufs[:5]
        exv = bufs[5:5 + hh_n]
        denv, srcc, dstc, cntv = bufs[5 + hh_n:]
        c = lax.axis_index("c")
        s = lax.axis_index("s")
        wid = c * NS + s
        eb = wid * EC
        eb2 = wid * ECP
        pltpu.sync_copy(ei_hbm.at[0, pl.ds(eb, EC)], srcv)
        pltpu.sync_copy(ei_hbm.at[1, pl.ds(eb, EC)], dstv)
        zero = jnp.zeros((L,), f32)
        zeroi = jnp.zeros((L,), i32)
        lane = lax.iota(i32, L)

        def phase(ph, _):
            b = ph // q_per_b
            pltpu.sync_copy(at_hbm.at[b, pl.ds(eb, EC)], attv)
            pltpu.sync_copy(es_hbm.at[ph], esv)
            pltpu.sync_copy(ed_hbm.at[ph], edv)

            def zbody(i, _):
                denv[pl.ds(i * L, L)] = zero
                return 0

            lax.fori_loop(0, hh_n * N // L, zbody, 0)

            def ebody(i, cnt):
                off = i * L
                sidx = srcv[pl.ds(off, L)]
                didx = dstv[pl.ds(off, L)]
                m = attv[pl.ds(off, L)] == 1
                cs = plsc.cumsum(jnp.where(m, 1, 0))
                pos = cnt + cs - 1
                plsc.store_scatter(srcc, [pos], sidx, mask=m)
                plsc.store_scatter(dstc, [pos], didx, mask=m)
                for hh in range(hh_n):
                    esg = plsc.load_gather(esv, [sidx + hh * N])
                    edg = plsc.load_gather(edv, [didx + hh * N])
                    e = esg + edg
                    e = jnp.where(e >= 0.0, e, e * 0.2)
                    ex = jnp.where(m, jnp.exp(e), 0.0)
                    plsc.store_scatter(exv[hh], [pos], ex, mask=m)
                    if hh_n == 1:
                        plsc.addupdate_scatter(denv, [didx], ex)
                    else:
                        plsc.addupdate_scatter(denv, [didx * hh_n + hh], ex)
                return cnt + cs[L - 1]

            cnt = lax.fori_loop(0, EC // L, ebody, 0)
            for kk in range(6):
                padix = cnt + kk * L + lane
                plsc.store_scatter(srcc, [padix], zeroi)
                plsc.store_scatter(dstc, [padix], zeroi)
                for hh in range(hh_n):
                    plsc.store_scatter(exv[hh], [padix], zero)
            cntv[pl.ds(0, L)] = jnp.full((L,), cnt, i32)
            for hh in range(hh_n):
                pltpu.sync_copy(exv[hh], ex_hbm.at[ph, hh, pl.ds(eb2, ECP)])
            pltpu.sync_copy(denv, den_hbm.at[ph, wid, pl.ds(0, hh_n * N)])
            pltpu.sync_copy(srcc, srcc_hbm.at[b, pl.ds(eb2, ECP)])
            pltpu.sync_copy(dstc, dstc_hbm.at[b, pl.ds(eb2, ECP)])
            pltpu.sync_copy(cntv, cnt_hbm.at[b, wid])
            return 0

        lax.fori_loop(0, nph, phase, 0)

    return k(ES, ED, ei, attrT)


def _sc_pass2(Tf, srcC, dstC, cnts, EX, zz, nph_outer, hh_n, trows, q_per_b):
    DW = 64
    VH = DW // L

    @functools.partial(
        pl.kernel,
        out_type=jax.ShapeDtypeStruct((NC, nph_outer * hh_n, NP, DW), f32),
        mesh=_MESH,
        scratch_types=[
            pltpu.VMEM((ECP,), i32),
            pltpu.VMEM((ECP,), i32),
            pltpu.VMEM((ECP,), f32),
            pltpu.VMEM((L,), i32),
            pltpu.VMEM((G, DW), f32),
            pltpu.VMEM((G, DW), f32),
            pltpu.VMEM((G, DW), f32),
            pltpu.VMEM((G, DW), f32),
            pltpu.VMEM((125, DW), f32),
            pltpu.VMEM_SHARED((N, DW), f32),
            pltpu.SemaphoreType.DMA,
            pltpu.SemaphoreType.DMA,
            pltpu.SemaphoreType.DMA,
            pltpu.SemaphoreType.DMA,
        ],
        compiler_params=pltpu.CompilerParams(use_tc_tiling_on_sc=False, needs_layout_passes=False),
    )
    def k(t_hbm, srcc_hbm, dstc_hbm, cnt_hbm, ex_hbm, zz_hbm, out_hbm,
          srcv, dstv, w0v, cntv, rb0, rb1, rb2, rb3, ztile, acc,
          rs0, rs1, rs2, rs3):
        rbufs = (rb0, rb1, rb2, rb3)
        rsems = (rs0, rs1, rs2, rs3)
        NBUF = 4
        c = lax.axis_index("c")
        s = lax.axis_index("s")
        wid = c * NS + s
        eb2 = wid * ECP
        nb0 = s * NSL
        pltpu.sync_copy(zz_hbm, ztile)
        for j in range(NSL // 125):
            pltpu.sync_copy(ztile, acc.at[pl.ds(nb0 + j * 125, 125)])
        plsc.subcore_barrier()

        def issue_gather(g, gb, gs, row_off):
            for kk in range(G // L):
                s16 = srcv[pl.ds(g * G + kk * L, L)]
                if hh_n == 2:
                    i16 = s16 * 2 + row_off
                else:
                    i16 = s16 + row_off
                pltpu.async_copy(t_hbm.at[i16], gb.at[pl.ds(kk * L, L)], gs)

        def drain_gather(gb, gs):
            for kk in range(G // L):
                pltpu.make_async_copy(
                    t_hbm.at[pl.ds(0, L)], gb.at[pl.ds(kk * L, L)], gs).wait()

        def drain_scatter(sb, ss):
            for kk in range(G // L):
                pltpu.make_async_copy(
                    sb.at[pl.ds(kk * L, L)], acc.at[pl.ds(0, L)], ss).wait()

        def process(g, gb, ss):
            base = g * G
            drain_gather(gb, ss)
            for kk in range(G // L):
                w0_16 = w0v[pl.ds(base + kk * L, L)]
                for rr in range(L):
                    r = kk * L + rr
                    w0 = w0_16[rr]
                    for v in range(VH):
                        gb[r, pl.ds(v * L, L)] = gb[r, pl.ds(v * L, L)] * w0
            for kk in range(G // L):
                d16 = dstv[pl.ds(base + kk * L, L)]
                pltpu.async_copy(gb.at[pl.ds(kk * L, L)], acc.at[d16], ss,
                                 add=True)

        def phase(ph, _):
            po = ph // hh_n
            hh = ph % hh_n
            row_off = po * (hh_n * trows) + hh
            b = ph // (hh_n * q_per_b)
            pltpu.sync_copy(cnt_hbm.at[b, wid], cntv)
            pltpu.sync_copy(srcc_hbm.at[b, pl.ds(eb2, ECP)], srcv)
            pltpu.sync_copy(dstc_hbm.at[b, pl.ds(eb2, ECP)], dstv)
            pltpu.sync_copy(ex_hbm.at[po, hh, pl.ds(eb2, ECP)], w0v)
            cnt = cntv[pl.ds(0, L)][0]
            ngd = (cnt + (G - 1)) // G
            nmac = (ngd + (NBUF - 1)) // NBUF + 1

            def slot_refill(g, kb):
                @pl.when(jnp.logical_and(g >= NBUF, g - NBUF < ngd))
                def _():
                    drain_scatter(rbufs[kb], rsems[kb])

                @pl.when(g < ngd)
                def _():
                    issue_gather(g, rbufs[kb], rsems[kb], row_off)

            def slot_process(g, kb):
                @pl.when(g < ngd)
                def _():
                    process(g, rbufs[kb], rsems[kb])

            @pl.when(ngd > 0)
            def _():
                issue_gather(0, rbufs[0], rsems[0], row_off)

            @pl.when(ngd > 1)
            def _():
                issue_gather(1, rbufs[1], rsems[1], row_off)

            def macro(m, _):
                g4 = m * NBUF
                slot_refill(g4 + 2, 2)
                slot_refill(g4 + 3, 3)
                slot_process(g4, 0)
                slot_process(g4 + 1, 1)
                slot_refill(g4 + 4, 0)
                slot_refill(g4 + 5, 1)
                slot_process(g4 + 2, 2)
                slot_process(g4 + 3, 3)
                return 0

            lax.fori_loop(0, nmac, macro, 0)
            plsc.subcore_barrier()
            pltpu.sync_copy(acc.at[pl.ds(nb0, NSL)],
                            out_hbm.at[c, ph, pl.ds(nb0, NSL)])
            for j in range(NSL // 125):
                pltpu.sync_copy(ztile, acc.at[pl.ds(nb0 + j * 125, 125)])
            plsc.subcore_barrier()
            return 0

        lax.fori_loop(0, nph_outer * hh_n, phase, 0)

    return k(Tf, srcC, dstC, cnts, EX, zz)


def _reduce_partials(DP, nph, inner):
    CH = 2048

    def body(dp_ref, out_ref):
        acc = dp_ref[:, 0]
        for kk in range(1, NW):
            acc = acc + dp_ref[:, kk]
        out_ref[...] = acc

    return pl.pallas_call(
        body,
        grid=(inner // CH,),
        in_specs=[pl.BlockSpec((nph, NW, CH), lambda i: (0, 0, i))],
        out_specs=pl.BlockSpec((nph, CH), lambda i: (0, i)),
        out_shape=jax.ShapeDtypeStruct((nph, inner), f32),
    )(DP)


def _phase_c(OUT1P, DEN1P, W2r, A2Sbd, A2Dbd):
    def body(p_ref, den_ref, w2_ref, a2s_ref, a2d_ref,
             t2_ref, es2_ref, ed2_ref):
        h2 = None
        for bq in range(NBQ):
            p0 = p_ref[0, 2 * bq] + p_ref[1, 2 * bq]
            p1 = p_ref[0, 2 * bq + 1] + p_ref[1, 2 * bq + 1]
            den = den_ref[bq]
            d0 = jnp.broadcast_to(den[:, 0:1], (BLKC, 64))
            d1 = jnp.broadcast_to(den[:, 1:2], (BLKC, 64))
            gacc = jnp.concatenate([p0 / (d0 + 1e-16), p1 / (d1 + 1e-16)],
                                   axis=1)
            gacc = jnp.where(gacc > 0, gacc, (jnp.exp(gacc) - 1.0))
            t = jnp.dot(gacc, w2_ref[bq], preferred_element_type=f32)
            h2 = t if h2 is None else h2 + t
        es2_ref[...] = jnp.dot(h2, a2s_ref[...], preferred_element_type=f32)
        ed2_ref[...] = jnp.dot(h2, a2d_ref[...], preferred_element_type=f32)
        for b in range(NB):
            t2_ref[b] = h2[:, b * 64:(b + 1) * 64]

    return pl.pallas_call(
        body,
        grid=(NP // BLKC,),
        in_specs=[
            pl.BlockSpec((NC, 2 * NBQ, BLKC, 64), lambda i: (0, 0, i, 0)),
            pl.BlockSpec((NBQ, BLKC, 2), lambda i: (0, i, 0)),
            pl.BlockSpec((NBQ, 128, NB * 64), lambda i: (0, 0, 0)),
            pl.BlockSpec((NB * 64, NB), lambda i: (0, 0)),
            pl.BlockSpec((NB * 64, NB), lambda i: (0, 0)),
        ],
        out_specs=[
            pl.BlockSpec((NB, BLKC, 64), lambda i: (0, i, 0)),
            pl.BlockSpec((BLKC, NB), lambda i: (i, 0)),
            pl.BlockSpec((BLKC, NB), lambda i: (i, 0)),
        ],
        out_shape=[
            jax.ShapeDtypeStruct((NB, NP, 64), f32),
            jax.ShapeDtypeStruct((NP, NB), f32),
            jax.ShapeDtypeStruct((NP, NB), f32),
        ],
    )(OUT1P, DEN1P, W2r, A2Sbd, A2Dbd)


def _phase_e(OUT2P, DEN2P, Wf, bf):
    def body(p_ref, den_ref, wf_ref, bf_ref, out_ref):
        ys = []
        for b in range(NB):
            p = p_ref[0, b] + p_ref[1, b]
            den = den_ref[b]
            d = jnp.broadcast_to(den, (BLKE, 64))
            y = p / (d + 1e-16)
            ys.append(jnp.where(y > 0, y, (jnp.exp(y) - 1.0)))
        y = jnp.concatenate(ys, axis=1)
        o = jnp.dot(y, wf_ref[...], preferred_element_type=f32) + bf_ref[...]
        out_ref[...] = jnp.where(o > 0, o, (jnp.exp(o) - 1.0))

    return pl.pallas_call(
        body,
        grid=(NP // BLKE,),
        in_specs=[
            pl.BlockSpec((NC, NB, BLKE, 64), lambda i: (0, 0, i, 0)),
            pl.BlockSpec((NB, BLKE, 1), lambda i: (0, i, 0)),
            pl.BlockSpec((NB * 64, 64), lambda i: (0, 0)),
            pl.BlockSpec((1, 64), lambda i: (0, 0)),
        ],
        out_specs=pl.BlockSpec((BLKE, 64), lambda i: (i, 0)),
        out_shape=jax.ShapeDtypeStruct((NP, 64), f32),
    )(OUT2P, DEN2P, Wf, bf)


def kernel(x, edge_index, edge_attr, W1, a1s, a1d, W2, a2s, a2d, Wf, bf):
    ei = edge_index.astype(i32)
    attrT = jnp.concatenate([edge_attr.astype(i32).T,
                             jnp.ones((1, E), i32)], axis=0)

    W1r = W1.reshape(NB, F_IN, 4, 128).transpose(0, 2, 1, 3)
    W1r = W1r.reshape(NBQ, F_IN, 128)
    a1s_r = a1s.reshape(NBQ, 2, 64)
    a1d_r = a1d.reshape(NBQ, 2, 64)
    A1Sq = jnp.zeros((NBQ, 128, 2), f32)
    A1Sq = A1Sq.at[:, 0:64, 0].set(a1s_r[:, 0, :])
    A1Sq = A1Sq.at[:, 64:128, 1].set(a1s_r[:, 1, :])
    A1Dq = jnp.zeros((NBQ, 128, 2), f32)
    A1Dq = A1Dq.at[:, 0:64, 0].set(a1d_r[:, 0, :])
    A1Dq = A1Dq.at[:, 64:128, 1].set(a1d_r[:, 1, :])
    W2r = jnp.zeros((NBQ, 128, NB * 64), f32)
    for b in range(NB):
        for q in range(4):
            W2r = W2r.at[b * 4 + q, :, b * 64:(b + 1) * 64].set(
                W2[b, q * 128:(q + 1) * 128, :])
    A2Sbd = jnp.zeros((NB * 64, NB), f32)
    A2Dbd = jnp.zeros((NB * 64, NB), f32)
    for b in range(NB):
        A2Sbd = A2Sbd.at[b * 64:(b + 1) * 64, b].set(a2s[b, 0, :])
        A2Dbd = A2Dbd.at[b * 64:(b + 1) * 64, b].set(a2d[b, 0, :])
    zz64 = jnp.zeros((125, 64), f32)

    T1, ES1, ED1 = _phase_a(x, W1r, A1Sq, A1Dq)
    ES1T = jnp.transpose(ES1, (0, 2, 1)).reshape(NBQ, 2 * N)
    ED1T = jnp.transpose(ED1, (0, 2, 1)).reshape(NBQ, 2 * N)
    EX1, DEN1P, SRCC, DSTC, CNTS = _sc_pass1(ES1T, ED1T, ei, attrT, NBQ, 2, 4)
    OUT1P = _sc_pass2(T1.reshape(NBQ * N * 2, 64), SRCC, DSTC, CNTS, EX1,
                      zz64, NBQ, 2, N, 4)
    DEN1S = _reduce_partials(DEN1P, NBQ, 2 * NP).reshape(NBQ, NP, 2)

    T2, ES2, ED2 = _phase_c(OUT1P, DEN1S, W2r, A2Sbd, A2Dbd)
    EX2, DEN2P, SRCC2, DSTC2, CNTS2 = _sc_pass1(ES2.T[:, :N], ED2.T[:, :N],
                                                ei, attrT, NB, 1, 1)
    OUT2P = _sc_pass2(T2.reshape(NB * NP, 64), SRCC2, DSTC2, CNTS2, EX2,
                      zz64, NB, 1, NP, 1)
    DEN2S = _reduce_partials(DEN2P, NB, NP).reshape(NB, NP, 1)

    return _phase_e(OUT2P, DEN2S, Wf, bf.reshape(1, 64))[:N]


# speedup vs baseline: 1.4513x; 1.0214x over previous
import functools

import jax
import jax.numpy as jnp
from jax import lax
from jax.experimental import pallas as pl
from jax.experimental.pallas import tpu as pltpu
from jax.experimental.pallas import tpu_sc as plsc

N = 10000
E = 320000
F_IN = 128
NB = 5
NBQ = 20
NC, NS, L = 2, 16, 16
NW = NC * NS
EC = E // NW
ECP = EC + 128
NSL = N // NS
G = 80
NG = EC // G
NP = 10240
BLKC = 256
BLKE = 512
f32 = jnp.float32
i32 = jnp.int32

_MESH = plsc.VectorSubcoreMesh(core_axis_name="c", subcore_axis_name="s",
                               num_cores=NC, num_subcores=NS)


def _phase_a(x, W1r, A1Sq, A1Dq):
    def body(x_ref, w_ref, as_ref, ad_ref, t1_ref, es_ref, ed_ref):
        h = jnp.dot(x_ref[...], w_ref[0], preferred_element_type=f32)
        t1_ref[0] = h
        es_ref[0] = jnp.dot(h, as_ref[0], preferred_element_type=f32)
        ed_ref[0] = jnp.dot(h, ad_ref[0], preferred_element_type=f32)

    return pl.pallas_call(
        body,
        grid=(NBQ,),
        in_specs=[
            pl.BlockSpec((N, F_IN), lambda k: (0, 0)),
            pl.BlockSpec((1, F_IN, 128), lambda k: (k, 0, 0)),
            pl.BlockSpec((1, 128, 2), lambda k: (k, 0, 0)),
            pl.BlockSpec((1, 128, 2), lambda k: (k, 0, 0)),
        ],
        out_specs=[
            pl.BlockSpec((1, N, 128), lambda k: (k, 0, 0)),
            pl.BlockSpec((1, N, 2), lambda k: (k, 0, 0)),
            pl.BlockSpec((1, N, 2), lambda k: (k, 0, 0)),
        ],
        out_shape=[
            jax.ShapeDtypeStruct((NBQ, N, 128), f32),
            jax.ShapeDtypeStruct((NBQ, N, 2), f32),
            jax.ShapeDtypeStruct((NBQ, N, 2), f32),
        ],
    )(x, W1r, A1Sq, A1Dq)


def _sc_pass1(ES, ED, ei, attrT, nph, hh_n, q_per_b):
    scratch = [
        pltpu.VMEM((EC,), i32),
        pltpu.VMEM((EC,), i32),
        pltpu.VMEM((EC,), i32),
        pltpu.VMEM((hh_n * N,), f32),
        pltpu.VMEM((hh_n * N,), f32),
    ]
    for _ in range(hh_n):
        scratch.append(pltpu.VMEM((ECP,), f32))
    scratch.append(pltpu.VMEM((hh_n * N,), f32))
    scratch.append(pltpu.VMEM((ECP,), i32))
    scratch.append(pltpu.VMEM((ECP,), i32))
    scratch.append(pltpu.VMEM((L,), i32))

    @functools.partial(
        pl.kernel,
        out_type=[
            jax.ShapeDtypeStruct((nph, hh_n, NW * ECP), f32),
            jax.ShapeDtypeStruct((nph, NW, hh_n * NP), f32),
            jax.ShapeDtypeStruct((NB, NW * ECP), i32),
            jax.ShapeDtypeStruct((NB, NW * ECP), i32),
            jax.ShapeDtypeStruct((NB, NW, L), i32),
        ],
        mesh=_MESH,
        scratch_types=scratch,
        compiler_params=pltpu.CompilerParams(use_tc_tiling_on_sc=False, needs_layout_passes=False),
    )
    def k(es_hbm, ed_hbm, ei_hbm, at_hbm,
          ex_hbm, den_hbm, srcc_hbm, dstc_hbm, cnt_hbm, *bufs):
        srcv, dstv, attv, esv, edv = bufs[:5]
        exv = bufs[5:5 + hh_n]
        denv, srcc, dstc, cntv = bufs[5 + hh_n:]
        c = lax.axis_index("c")
        s = lax.axis_index("s")
        wid = c * NS + s
        eb = wid * EC
        eb2 = wid * ECP
        pltpu.sync_copy(ei_hbm.at[0, pl.ds(eb, EC)], srcv)
        pltpu.sync_copy(ei_hbm.at[1, pl.ds(eb, EC)], dstv)
        zero = jnp.zeros((L,), f32)
        zeroi = jnp.zeros((L,), i32)
        lane = lax.iota(i32, L)

        def phase(ph, _):
            b = ph // q_per_b

            @pl.when(ph % q_per_b == 0)
            def _():
                pltpu.sync_copy(at_hbm.at[b, pl.ds(eb, EC)], attv)

            pltpu.sync_copy(es_hbm.at[ph], esv)
            pltpu.sync_copy(ed_hbm.at[ph], edv)

            def zbody(i, _):
                denv[pl.ds(i * L, L)] = zero
                return 0

            lax.fori_loop(0, hh_n * N // L, zbody, 0)

            def ebody(i, cnt):
                off = i * L
                sidx = srcv[pl.ds(off, L)]
                didx = dstv[pl.ds(off, L)]
                m = attv[pl.ds(off, L)] == 1
                cs = plsc.cumsum(jnp.where(m, 1, 0))
                pos = cnt + cs - 1
                plsc.store_scatter(srcc, [pos], sidx, mask=m)
                plsc.store_scatter(dstc, [pos], didx, mask=m)
                for hh in range(hh_n):
                    esg = plsc.load_gather(esv, [sidx + hh * N])
                    edg = plsc.load_gather(edv, [didx + hh * N])
                    e = esg + edg
                    e = jnp.where(e >= 0.0, e, e * 0.2)
                    ex = jnp.where(m, jnp.exp(e), 0.0)
                    plsc.store_scatter(exv[hh], [pos], ex, mask=m)
                    if hh_n == 1:
                        plsc.addupdate_scatter(denv, [didx], ex)
                    else:
                        plsc.addupdate_scatter(denv, [didx * hh_n + hh], ex)
                return cnt + cs[L - 1]

            cnt = lax.fori_loop(0, EC // L, ebody, 0)
            for kk in range(6):
                padix = cnt + kk * L + lane
                plsc.store_scatter(srcc, [padix], zeroi)
                plsc.store_scatter(dstc, [padix], zeroi)
                for hh in range(hh_n):
                    plsc.store_scatter(exv[hh], [padix], zero)
            cntv[pl.ds(0, L)] = jnp.full((L,), cnt, i32)
            for hh in range(hh_n):
                pltpu.sync_copy(exv[hh], ex_hbm.at[ph, hh, pl.ds(eb2, ECP)])
            pltpu.sync_copy(denv, den_hbm.at[ph, wid, pl.ds(0, hh_n * N)])
            pltpu.sync_copy(srcc, srcc_hbm.at[b, pl.ds(eb2, ECP)])
            pltpu.sync_copy(dstc, dstc_hbm.at[b, pl.ds(eb2, ECP)])
            pltpu.sync_copy(cntv, cnt_hbm.at[b, wid])
            return 0

        lax.fori_loop(0, nph, phase, 0)

    return k(ES, ED, ei, attrT)


def _sc_pass2(Tf, srcC, dstC, cnts, EX, zz, nph_outer, hh_n, trows, q_per_b):
    DW = 64
    VH = DW // L

    @functools.partial(
        pl.kernel,
        out_type=jax.ShapeDtypeStruct((NC, nph_outer * hh_n, NP, DW), f32),
        mesh=_MESH,
        scratch_types=[
            pltpu.VMEM((ECP,), i32),
            pltpu.VMEM((ECP,), i32),
            pltpu.VMEM((ECP,), f32),
            pltpu.VMEM((L,), i32),
            pltpu.VMEM((G, DW), f32),
            pltpu.VMEM((G, DW), f32),
            pltpu.VMEM((G, DW), f32),
            pltpu.VMEM((G, DW), f32),
            pltpu.VMEM((125, DW), f32),
            pltpu.VMEM_SHARED((N, DW), f32),
            pltpu.SemaphoreType.DMA,
            pltpu.SemaphoreType.DMA,
            pltpu.SemaphoreType.DMA,
            pltpu.SemaphoreType.DMA,
        ],
        compiler_params=pltpu.CompilerParams(use_tc_tiling_on_sc=False, needs_layout_passes=False),
    )
    def k(t_hbm, srcc_hbm, dstc_hbm, cnt_hbm, ex_hbm, zz_hbm, out_hbm,
          srcv, dstv, w0v, cntv, rb0, rb1, rb2, rb3, ztile, acc,
          rs0, rs1, rs2, rs3):
        rbufs = (rb0, rb1, rb2, rb3)
        rsems = (rs0, rs1, rs2, rs3)
        NBUF = 4
        c = lax.axis_index("c")
        s = lax.axis_index("s")
        wid = c * NS + s
        eb2 = wid * ECP
        nb0 = s * NSL
        pltpu.sync_copy(zz_hbm, ztile)
        for j in range(NSL // 125):
            pltpu.sync_copy(ztile, acc.at[pl.ds(nb0 + j * 125, 125)])
        plsc.subcore_barrier()

        def issue_gather(g, gb, gs, row_off):
            for kk in range(G // L):
                s16 = srcv[pl.ds(g * G + kk * L, L)]
                if hh_n == 2:
                    i16 = s16 * 2 + row_off
                else:
                    i16 = s16 + row_off
                pltpu.async_copy(t_hbm.at[i16], gb.at[pl.ds(kk * L, L)], gs)

        def drain_gather(gb, gs):
            for kk in range(G // L):
                pltpu.make_async_copy(
                    t_hbm.at[pl.ds(0, L)], gb.at[pl.ds(kk * L, L)], gs).wait()

        def drain_scatter(sb, ss):
            for kk in range(G // L):
                pltpu.make_async_copy(
                    sb.at[pl.ds(kk * L, L)], acc.at[pl.ds(0, L)], ss).wait()

        def process(g, gb, ss):
            base = g * G
            drain_gather(gb, ss)
            for kk in range(G // L):
                w0_16 = w0v[pl.ds(base + kk * L, L)]
                for rr in range(L):
                    r = kk * L + rr
                    w0 = w0_16[rr]
                    for v in range(VH):
                        gb[r, pl.ds(v * L, L)] = gb[r, pl.ds(v * L, L)] * w0
            for kk in range(G // L):
                d16 = dstv[pl.ds(base + kk * L, L)]
                pltpu.async_copy(gb.at[pl.ds(kk * L, L)], acc.at[d16], ss,
                                 add=True)

        def phase(ph, _):
            po = ph // hh_n
            hh = ph % hh_n
            row_off = po * (hh_n * trows) + hh
            b = ph // (hh_n * q_per_b)

            @pl.when(ph % (hh_n * q_per_b) == 0)
            def _():
                pltpu.sync_copy(cnt_hbm.at[b, wid], cntv)
                pltpu.sync_copy(srcc_hbm.at[b, pl.ds(eb2, ECP)], srcv)
                pltpu.sync_copy(dstc_hbm.at[b, pl.ds(eb2, ECP)], dstv)

            pltpu.sync_copy(ex_hbm.at[po, hh, pl.ds(eb2, ECP)], w0v)
            cnt = cntv[pl.ds(0, L)][0]
            ngd = (cnt + (G - 1)) // G
            nmac = (ngd + (NBUF - 1)) // NBUF + 1

            def slot_refill(g, kb):
                @pl.when(jnp.logical_and(g >= NBUF, g - NBUF < ngd))
                def _():
                    drain_scatter(rbufs[kb], rsems[kb])

                @pl.when(g < ngd)
                def _():
                    issue_gather(g, rbufs[kb], rsems[kb], row_off)

            def slot_process(g, kb):
                @pl.when(g < ngd)
                def _():
                    process(g, rbufs[kb], rsems[kb])

            @pl.when(ngd > 0)
            def _():
                issue_gather(0, rbufs[0], rsems[0], row_off)

            @pl.when(ngd > 1)
            def _():
                issue_gather(1, rbufs[1], rsems[1], row_off)

            def macro(m, _):
                g4 = m * NBUF
                slot_refill(g4 + 2, 2)
                slot_refill(g4 + 3, 3)
                slot_process(g4, 0)
                slot_process(g4 + 1, 1)
                slot_refill(g4 + 4, 0)
                slot_refill(g4 + 5, 1)
                slot_process(g4 + 2, 2)
                slot_process(g4 + 3, 3)
                return 0

            lax.fori_loop(0, nmac, macro, 0)
            plsc.subcore_barrier()
            pltpu.sync_copy(acc.at[pl.ds(nb0, NSL)],
                            out_hbm.at[c, ph, pl.ds(nb0, NSL)])
            for j in range(NSL // 125):
                pltpu.sync_copy(ztile, acc.at[pl.ds(nb0 + j * 125, 125)])
            plsc.subcore_barrier()
            return 0

        lax.fori_loop(0, nph_outer * hh_n, phase, 0)

    return k(Tf, srcC, dstC, cnts, EX, zz)


def _reduce_partials(DP, nph, inner):
    CH = 2048

    def body(dp_ref, out_ref):
        acc = dp_ref[:, 0]
        for kk in range(1, NW):
            acc = acc + dp_ref[:, kk]
        out_ref[...] = acc

    return pl.pallas_call(
        body,
        grid=(inner // CH,),
        in_specs=[pl.BlockSpec((nph, NW, CH), lambda i: (0, 0, i))],
        out_specs=pl.BlockSpec((nph, CH), lambda i: (0, i)),
        out_shape=jax.ShapeDtypeStruct((nph, inner), f32),
    )(DP)


def _phase_c(OUT1P, DEN1P, W2r, A2Sbd, A2Dbd):
    def body(p_ref, den_ref, w2_ref, a2s_ref, a2d_ref,
             t2_ref, es2_ref, ed2_ref):
        h2 = None
        for bq in range(NBQ):
            p0 = p_ref[0, 2 * bq] + p_ref[1, 2 * bq]
            p1 = p_ref[0, 2 * bq + 1] + p_ref[1, 2 * bq + 1]
            den = den_ref[bq]
            d0 = jnp.broadcast_to(den[:, 0:1], (BLKC, 64))
            d1 = jnp.broadcast_to(den[:, 1:2], (BLKC, 64))
            gacc = jnp.concatenate([p0 / (d0 + 1e-16), p1 / (d1 + 1e-16)],
                                   axis=1)
            gacc = jnp.where(gacc > 0, gacc, (jnp.exp(gacc) - 1.0))
            t = jnp.dot(gacc, w2_ref[bq], preferred_element_type=f32)
            h2 = t if h2 is None else h2 + t
        es2_ref[...] = jnp.dot(h2, a2s_ref[...], preferred_element_type=f32)
        ed2_ref[...] = jnp.dot(h2, a2d_ref[...], preferred_element_type=f32)
        for b in range(NB):
            t2_ref[b] = h2[:, b * 64:(b + 1) * 64]

    return pl.pallas_call(
        body,
        grid=(NP // BLKC,),
        in_specs=[
            pl.BlockSpec((NC, 2 * NBQ, BLKC, 64), lambda i: (0, 0, i, 0)),
            pl.BlockSpec((NBQ, BLKC, 2), lambda i: (0, i, 0)),
            pl.BlockSpec((NBQ, 128, NB * 64), lambda i: (0, 0, 0)),
            pl.BlockSpec((NB * 64, NB), lambda i: (0, 0)),
            pl.BlockSpec((NB * 64, NB), lambda i: (0, 0)),
        ],
        out_specs=[
            pl.BlockSpec((NB, BLKC, 64), lambda i: (0, i, 0)),
            pl.BlockSpec((BLKC, NB), lambda i: (i, 0)),
            pl.BlockSpec((BLKC, NB), lambda i: (i, 0)),
        ],
        out_shape=[
            jax.ShapeDtypeStruct((NB, NP, 64), f32),
            jax.ShapeDtypeStruct((NP, NB), f32),
            jax.ShapeDtypeStruct((NP, NB), f32),
        ],
    )(OUT1P, DEN1P, W2r, A2Sbd, A2Dbd)


def _phase_e(OUT2P, DEN2P, Wf, bf):
    def body(p_ref, den_ref, wf_ref, bf_ref, out_ref):
        ys = []
        for b in range(NB):
            p = p_ref[0, b] + p_ref[1, b]
            den = den_ref[b]
            d = jnp.broadcast_to(den, (BLKE, 64))
            y = p / (d + 1e-16)
            ys.append(jnp.where(y > 0, y, (jnp.exp(y) - 1.0)))
        y = jnp.concatenate(ys, axis=1)
        o = jnp.dot(y, wf_ref[...], preferred_element_type=f32) + bf_ref[...]
        out_ref[...] = jnp.where(o > 0, o, (jnp.exp(o) - 1.0))

    return pl.pallas_call(
        body,
        grid=(NP // BLKE,),
        in_specs=[
            pl.BlockSpec((NC, NB, BLKE, 64), lambda i: (0, 0, i, 0)),
            pl.BlockSpec((NB, BLKE, 1), lambda i: (0, i, 0)),
            pl.BlockSpec((NB * 64, 64), lambda i: (0, 0)),
            pl.BlockSpec((1, 64), lambda i: (0, 0)),
        ],
        out_specs=pl.BlockSpec((BLKE, 64), lambda i: (i, 0)),
        out_shape=jax.ShapeDtypeStruct((NP, 64), f32),
    )(OUT2P, DEN2P, Wf, bf)


def kernel(x, edge_index, edge_attr, W1, a1s, a1d, W2, a2s, a2d, Wf, bf):
    ei = edge_index.astype(i32)
    attrT = jnp.concatenate([edge_attr.astype(i32).T,
                             jnp.ones((1, E), i32)], axis=0)

    W1r = W1.reshape(NB, F_IN, 4, 128).transpose(0, 2, 1, 3)
    W1r = W1r.reshape(NBQ, F_IN, 128)
    a1s_r = a1s.reshape(NBQ, 2, 64)
    a1d_r = a1d.reshape(NBQ, 2, 64)
    A1Sq = jnp.zeros((NBQ, 128, 2), f32)
    A1Sq = A1Sq.at[:, 0:64, 0].set(a1s_r[:, 0, :])
    A1Sq = A1Sq.at[:, 64:128, 1].set(a1s_r[:, 1, :])
    A1Dq = jnp.zeros((NBQ, 128, 2), f32)
    A1Dq = A1Dq.at[:, 0:64, 0].set(a1d_r[:, 0, :])
    A1Dq = A1Dq.at[:, 64:128, 1].set(a1d_r[:, 1, :])
    W2r = jnp.zeros((NBQ, 128, NB * 64), f32)
    for b in range(NB):
        for q in range(4):
            W2r = W2r.at[b * 4 + q, :, b * 64:(b + 1) * 64].set(
                W2[b, q * 128:(q + 1) * 128, :])
    A2Sbd = jnp.zeros((NB * 64, NB), f32)
    A2Dbd = jnp.zeros((NB * 64, NB), f32)
    for b in range(NB):
        A2Sbd = A2Sbd.at[b * 64:(b + 1) * 64, b].set(a2s[b, 0, :])
        A2Dbd = A2Dbd.at[b * 64:(b + 1) * 64, b].set(a2d[b, 0, :])
    zz64 = jnp.zeros((125, 64), f32)

    T1, ES1, ED1 = _phase_a(x, W1r, A1Sq, A1Dq)
    ES1T = jnp.transpose(ES1, (0, 2, 1)).reshape(NBQ, 2 * N)
    ED1T = jnp.transpose(ED1, (0, 2, 1)).reshape(NBQ, 2 * N)
    EX1, DEN1P, SRCC, DSTC, CNTS = _sc_pass1(ES1T, ED1T, ei, attrT, NBQ, 2, 4)
    OUT1P = _sc_pass2(T1.reshape(NBQ * N * 2, 64), SRCC, DSTC, CNTS, EX1,
                      zz64, NBQ, 2, N, 4)
    DEN1S = _reduce_partials(DEN1P, NBQ, 2 * NP).reshape(NBQ, NP, 2)

    T2, ES2, ED2 = _phase_c(OUT1P, DEN1S, W2r, A2Sbd, A2Dbd)
    EX2, DEN2P, SRCC2, DSTC2, CNTS2 = _sc_pass1(ES2.T[:, :N], ED2.T[:, :N],
                                                ei, attrT, NB, 1, 1)
    OUT2P = _sc_pass2(T2.reshape(NB * NP, 64), SRCC2, DSTC2, CNTS2, EX2,
                      zz64, NB, 1, NP, 1)
    DEN2S = _reduce_partials(DEN2P, NB, NP).reshape(NB, NP, 1)

    return _phase_e(OUT2P, DEN2S, Wf, bf.reshape(1, 64))[:N]
